# baseline (speedup 1.0000x reference)
import functools

import jax
import jax.numpy as jnp
import numpy as np
from jax.experimental import pallas as pl

_BLOCK = 8192
_KS0 = np.uint32(0)
_KS1 = np.uint32(42)
_KS2 = np.uint32(0x1BD11BDA) ^ np.uint32(42)
_ROTS_A = (13, 15, 26, 6)
_ROTS_B = (17, 29, 16, 24)
_TINY = np.float32(np.finfo(np.float32).tiny)
_ONE_BITS = np.uint32(0x3F800000)


def _rotl(x, r):
    return (x << np.uint32(r)) | (x >> np.uint32(32 - r))


def _threefry_bits(c1):
    ks = (_KS0, _KS1, _KS2)
    x0 = jnp.zeros_like(c1)
    x1 = c1 + _KS1
    for group in range(5):
        rots = _ROTS_A if group % 2 == 0 else _ROTS_B
        for r in rots:
            x0 = x0 + x1
            x1 = _rotl(x1, r)
            x1 = x1 ^ x0
        x0 = x0 + ks[(group + 1) % 3]
        x1 = x1 + ks[(group + 2) % 3] + np.uint32(group + 1)
    return x0 ^ x1


def _sample_kernel(x_ref, val_ref, idx_ref, *, vocab, block):
    j = pl.program_id(0)

    @pl.when(j == 0)
    def _init():
        val_ref[...] = jnp.full(val_ref.shape, -jnp.inf, val_ref.dtype)
        idx_ref[...] = jnp.zeros(idx_ref.shape, idx_ref.dtype)

    rows, b = x_ref.shape
    col = jax.lax.broadcasted_iota(jnp.int32, (rows, b), 1) + j * block
    row = jax.lax.broadcasted_iota(jnp.int32, (rows, b), 0)
    flat = (row * vocab + col).astype(jnp.uint32)

    bits = _threefry_bits(flat)
    float_bits = (bits >> np.uint32(9)) | _ONE_BITS
    floats = jax.lax.bitcast_convert_type(float_bits, jnp.float32)
    floats = floats - np.float32(1.0)
    u = jnp.maximum(floats * (np.float32(1.0) - _TINY) + _TINY, _TINY)
    gumbel = -jnp.log(-jnp.log(u))

    score = x_ref[...] + gumbel
    score = jnp.where(col < vocab, score, -jnp.inf)

    block_val = jnp.max(score, axis=1, keepdims=True)
    cand = jnp.where(score == block_val, col, jnp.int32(np.iinfo(np.int32).max))
    block_idx = jnp.min(cand, axis=1, keepdims=True)

    better = block_val > val_ref[...]
    val_ref[...] = jnp.where(better, block_val, val_ref[...])
    idx_ref[...] = jnp.where(better, block_idx, idx_ref[...])


@jax.jit
def kernel(outputs):
    rows, vocab = outputs.shape
    grid = (pl.cdiv(vocab, _BLOCK),)
    _, idx = pl.pallas_call(
        functools.partial(_sample_kernel, vocab=vocab, block=_BLOCK),
        grid=grid,
        in_specs=[pl.BlockSpec((rows, _BLOCK), lambda j: (0, j))],
        out_specs=[
            pl.BlockSpec((rows, 1), lambda j: (0, 0)),
            pl.BlockSpec((rows, 1), lambda j: (0, 0)),
        ],
        out_shape=[
            jax.ShapeDtypeStruct((rows, 1), jnp.float32),
            jax.ShapeDtypeStruct((rows, 1), jnp.int32),
        ],
    )(outputs)
    return idx[:, 0]


# speedup vs baseline: 1.5195x; 1.5195x over previous
import functools

import jax
import jax.numpy as jnp
import numpy as np
from jax.experimental import pallas as pl
from jax.experimental.pallas import tpu as pltpu

_BLOCK = 8192
_CHUNK = 512
_KS0 = np.uint32(0)
_KS1 = np.uint32(42)
_KS2 = np.uint32(0x1BD11BDA) ^ np.uint32(42)
_ROTS = (13, 15, 26, 6, 17, 29, 16, 24, 13, 15, 26, 6, 17, 29, 16, 24, 13, 15, 26, 6)
_TINY = np.float32(np.finfo(np.float32).tiny)
_ONE_BITS = np.uint32(0x3F800000)
_INT_MAX = np.int32(np.iinfo(np.int32).max)


def _rotl(x, r):
    return (x << np.uint32(r)) | (x >> np.uint32(32 - r))


def _threefry_bits(c1):
    ks = (_KS0, _KS1, _KS2)
    x0 = c1
    x1 = _rotl(c1, _ROTS[0]) ^ x0
    for i, r in enumerate(_ROTS[1:], start=1):
        x0 = x0 + x1
        x1 = _rotl(x1, r) ^ x0
        if i % 4 == 3:
            group = i // 4
            kx = ks[(group + 1) % 3]
            ky = np.uint32(ks[(group + 2) % 3] + np.uint32(group + 1))
            if kx != _KS0:
                x0 = x0 + kx
            x1 = x1 + ky
    return x0 ^ x1


def _sample_kernel(x_ref, val_ref, idx_ref, acc_val, acc_col, *, vocab, block,
                   chunk, ngrid):
    j = pl.program_id(0)
    rows = x_ref.shape[0]

    @pl.when(j == 0)
    def _init():
        acc_val[...] = jnp.full(acc_val.shape, -jnp.inf, acc_val.dtype)
        acc_col[...] = jnp.zeros(acc_col.shape, acc_col.dtype)

    run_val = acc_val[...]
    run_col = acc_col[...]

    lane = jax.lax.broadcasted_iota(jnp.int32, (rows, chunk), 1)
    row = jax.lax.broadcasted_iota(jnp.int32, (rows, chunk), 0)
    seed_base = (row * vocab + lane + 42).astype(jnp.uint32)
    base0 = j * block

    for c in range(block // chunk):
        off = c * chunk
        x = x_ref[:, off:off + chunk]
        c1 = seed_base + jnp.uint32(base0 + off)
        bits = _threefry_bits(c1)
        float_bits = (bits >> np.uint32(9)) | _ONE_BITS
        floats = jax.lax.bitcast_convert_type(float_bits, jnp.float32)
        u = jnp.maximum(floats - np.float32(1.0), _TINY)
        score = x + (-jnp.log(-jnp.log(u)))
        col = lane + (base0 + off)
        score = jnp.where(col < vocab, score, -jnp.inf)
        upd = score > run_val
        run_val = jnp.maximum(run_val, score)
        run_col = jnp.where(upd, col, run_col)

    acc_val[...] = run_val
    acc_col[...] = run_col

    @pl.when(j == ngrid - 1)
    def _final():
        best = jnp.max(run_val, axis=1, keepdims=True)
        cand = jnp.where(run_val == best, run_col, _INT_MAX)
        val_ref[...] = best
        idx_ref[...] = jnp.min(cand, axis=1, keepdims=True)


@jax.jit
def kernel(outputs):
    rows, vocab = outputs.shape
    ngrid = pl.cdiv(vocab, _BLOCK)
    _, idx = pl.pallas_call(
        functools.partial(_sample_kernel, vocab=vocab, block=_BLOCK,
                          chunk=_CHUNK, ngrid=ngrid),
        grid=(ngrid,),
        in_specs=[pl.BlockSpec((rows, _BLOCK), lambda j: (0, j))],
        out_specs=[
            pl.BlockSpec((rows, 1), lambda j: (0, 0)),
            pl.BlockSpec((rows, 1), lambda j: (0, 0)),
        ],
        out_shape=[
            jax.ShapeDtypeStruct((rows, 1), jnp.float32),
            jax.ShapeDtypeStruct((rows, 1), jnp.int32),
        ],
        scratch_shapes=[
            pltpu.VMEM((rows, _CHUNK), jnp.float32),
            pltpu.VMEM((rows, _CHUNK), jnp.int32),
        ],
    )(outputs)
    return idx[:, 0]


# speedup vs baseline: 1.5479x; 1.0186x over previous
import functools

import jax
import jax.numpy as jnp
import numpy as np
from jax.experimental import pallas as pl
from jax.experimental.pallas import tpu as pltpu

_BLOCK = 16384
_CHUNK = 512
_KS0 = np.uint32(0)
_KS1 = np.uint32(42)
_KS2 = np.uint32(0x1BD11BDA) ^ np.uint32(42)
_ROTS = (13, 15, 26, 6, 17, 29, 16, 24, 13, 15, 26, 6, 17, 29, 16, 24, 13, 15, 26, 6)
_TINY = np.float32(np.finfo(np.float32).tiny)
_ONE_BITS = np.uint32(0x3F800000)
_INT_MAX = np.int32(np.iinfo(np.int32).max)


def _rotl(x, r):
    return (x << np.uint32(r)) | (x >> np.uint32(32 - r))


def _threefry_bits(c1):
    ks = (_KS0, _KS1, _KS2)
    x0 = c1
    x1 = _rotl(c1, _ROTS[0]) ^ x0
    for i, r in enumerate(_ROTS[1:], start=1):
        x0 = x0 + x1
        x1 = _rotl(x1, r) ^ x0
        if i % 4 == 3:
            group = i // 4
            kx = ks[(group + 1) % 3]
            ky = np.uint32(ks[(group + 2) % 3] + np.uint32(group + 1))
            if kx != _KS0:
                x0 = x0 + kx
            x1 = x1 + ky
    return x0 ^ x1


def _chunk_update(x, lane, seed_base, base, run_val, run_col, *, vocab, masked):
    c1 = seed_base + jnp.uint32(base)
    bits = _threefry_bits(c1)
    float_bits = (bits >> np.uint32(9)) | _ONE_BITS
    floats = jax.lax.bitcast_convert_type(float_bits, jnp.float32)
    u = jnp.maximum(floats - np.float32(1.0), _TINY)
    score = x + (-jnp.log(-jnp.log(u)))
    col = lane + base
    if masked:
        score = jnp.where(col < vocab, score, -jnp.inf)
    upd = score > run_val
    run_val = jnp.maximum(run_val, score)
    run_col = jnp.where(upd, col, run_col)
    return run_val, run_col


def _sample_kernel(x_ref, val_ref, idx_ref, acc_val, acc_col, *, vocab, block,
                   chunk, ngrid):
    j = pl.program_id(0)
    rows = x_ref.shape[0]

    @pl.when(j == 0)
    def _init():
        acc_val[...] = jnp.full(acc_val.shape, -jnp.inf, acc_val.dtype)
        acc_col[...] = jnp.zeros(acc_col.shape, acc_col.dtype)

    lane = jax.lax.broadcasted_iota(jnp.int32, (rows, chunk), 1)
    row = jax.lax.broadcasted_iota(jnp.int32, (rows, chunk), 0)
    seed_base = (row * vocab + lane + 42).astype(jnp.uint32)
    base0 = j * block

    tail_cols = vocab - (ngrid - 1) * block
    tail_chunks = pl.cdiv(tail_cols, chunk)

    @pl.when(j < ngrid - 1)
    def _full_block():
        run_val = acc_val[...]
        run_col = acc_col[...]
        for c in range(block // chunk):
            off = c * chunk
            run_val, run_col = _chunk_update(
                x_ref[:, off:off + chunk], lane, seed_base, base0 + off,
                run_val, run_col, vocab=vocab, masked=False)
        acc_val[...] = run_val
        acc_col[...] = run_col

    @pl.when(j == ngrid - 1)
    def _tail_block():
        run_val = acc_val[...]
        run_col = acc_col[...]
        for c in range(tail_chunks):
            off = c * chunk
            run_val, run_col = _chunk_update(
                x_ref[:, off:off + chunk], lane, seed_base, base0 + off,
                run_val, run_col, vocab=vocab,
                masked=(tail_cols - off) < chunk)
        best = jnp.max(run_val, axis=1, keepdims=True)
        cand = jnp.where(run_val == best, run_col, _INT_MAX)
        val_ref[...] = best
        idx_ref[...] = jnp.min(cand, axis=1, keepdims=True)


@jax.jit
def kernel(outputs):
    rows, vocab = outputs.shape
    ngrid = pl.cdiv(vocab, _BLOCK)
    _, idx = pl.pallas_call(
        functools.partial(_sample_kernel, vocab=vocab, block=_BLOCK,
                          chunk=_CHUNK, ngrid=ngrid),
        grid=(ngrid,),
        in_specs=[pl.BlockSpec((rows, _BLOCK), lambda j: (0, j))],
        out_specs=[
            pl.BlockSpec((rows, 1), lambda j: (0, 0)),
            pl.BlockSpec((rows, 1), lambda j: (0, 0)),
        ],
        out_shape=[
            jax.ShapeDtypeStruct((rows, 1), jnp.float32),
            jax.ShapeDtypeStruct((rows, 1), jnp.int32),
        ],
        scratch_shapes=[
            pltpu.VMEM((rows, _CHUNK), jnp.float32),
            pltpu.VMEM((rows, _CHUNK), jnp.int32),
        ],
    )(outputs)
    return idx[:, 0]
